# baseline (speedup 1.0000x reference)
import functools

import jax
import jax.numpy as jnp
from jax import lax
from jax.experimental import pallas as pl
from jax.experimental.pallas import tpu as pltpu
from jax.experimental.pallas import tpu_sc as plsc

_N = 10000
_E = 320000
_D = 128
_R = 16

_NC = 2
_NS = 16
_NW = _NC * _NS
_C = 112
_NCHUNK = 90
_K0 = 90
_K1 = 2 * _NCHUNK - _K0
_EPW = _NCHUNK * _C
_EPAD = 2 * _NCHUNK * _C * _NS
_NR = 3
_NP = 6
_NPAD = 10112
_RZ = _NPAD // _NS


def _mm_body(h_ref, w_ref, y_ref):
    y_ref[0] = jnp.dot(h_ref[...], w_ref[0], preferred_element_type=jnp.float32)


def _relation_matmul(node_features, weight):
    bn = 2000
    return pl.pallas_call(
        _mm_body,
        grid=(_N // bn, _R),
        in_specs=[
            pl.BlockSpec((bn, _D), lambda i, r: (i, 0)),
            pl.BlockSpec((1, _D, _D), lambda i, r: (r, 0, 0)),
        ],
        out_specs=pl.BlockSpec((1, bn, _D), lambda i, r: (r, i, 0)),
        out_shape=jax.ShapeDtypeStruct((_R, _N, _D), jnp.float32),
    )(node_features, weight)


def _sc_body(pack_hbm, norm_hbm, y_hbm, zeros_hbm, out_hbm,
             pack_v, norm_v, rows_v, acc, *sems):
    cid = lax.axis_index("c")
    sid = lax.axis_index("s")
    base = sid * (2 * _NCHUNK) + cid * _K0
    nself = lax.select(cid == 0, jnp.int32(_K0), jnp.int32(_K1))
    lsem = sems[0:_NP]
    gsem = sems[_NP:_NP + _NR]
    ssem = sems[_NP + _NR:_NP + 2 * _NR]

    pltpu.sync_copy(zeros_hbm, acc.at[pl.ds(sid * _RZ, _RZ)])
    plsc.subcore_barrier()

    def start_load(p, k):
        pltpu.async_copy(pack_hbm.at[base + k], pack_v.at[p], lsem[p])
        pltpu.async_copy(norm_hbm.at[base + k], norm_v.at[p], lsem[p])

    def wait_load(p):
        pltpu.make_async_copy(pack_hbm.at[0], pack_v.at[p], lsem[p]).wait()
        pltpu.make_async_copy(norm_hbm.at[0], norm_v.at[p], lsem[p]).wait()

    def start_gather(s, p):
        pltpu.async_copy(y_hbm.at[pack_v.at[p, 0]], rows_v.at[s], gsem[s])

    def wait_gather(s):
        pltpu.make_async_copy(y_hbm.at[pl.ds(0, _C)], rows_v.at[s], gsem[s]).wait()

    def start_scatter(s, p):
        pltpu.async_copy(rows_v.at[s], acc.at[pack_v.at[p, 1]], ssem[s], add=True)

    def wait_scatter(s):
        pltpu.make_async_copy(y_hbm.at[pl.ds(0, _C)], rows_v.at[s], ssem[s]).wait()

    def scale(s, p):
        def scale_body(i, carry):
            for u in range(4):
                e = i * 4 + u
                nv = plsc.load_gather(
                    norm_v,
                    [jnp.full((16,), p, jnp.int32), jnp.full((16,), e, jnp.int32)],
                )
                for j in range(_D // 16):
                    sl = pl.ds(j * 16, 16)
                    rows_v[s, e, sl] = rows_v[s, e, sl] * nv
            return carry

        lax.fori_loop(0, _C // 4, scale_body, 0)

    for k in range(4):
        start_load(k, k)
    wait_load(0)
    start_gather(0, 0)

    for k in range(6):
        s, p = k % _NR, k % _NP
        s1, p1 = (k + 1) % _NR, (k + 1) % _NP
        if k >= 2:
            wait_scatter(s1)
        wait_load(p1)
        start_gather(s1, p1)
        start_load((k + 4) % _NP, k + 4)
        wait_gather(s)
        scale(s, p)
        start_scatter(s, p)

    def six_body(q, carry):
        for u in range(6):
            k = q * 6 + u
            s, p = u % _NR, u
            s1, p1 = (u + 1) % _NR, (u + 1) % _NP

            @pl.when(k + 1 < nself)
            def _():
                wait_scatter(s1)
                wait_load(p1)
                start_gather(s1, p1)

            @pl.when(k + 4 < nself)
            def _():
                start_load((u + 4) % _NP, k + 4)

            wait_gather(s)
            scale(s, p)
            start_scatter(s, p)
        return carry

    lax.fori_loop(1, nself // 6, six_body, 0)

    for s in range(_NR):
        wait_scatter(s)

    plsc.subcore_barrier()

    last_full = _N // _RZ
    rem = _N - last_full * _RZ

    @pl.when(sid < last_full)
    def _():
        pltpu.sync_copy(
            acc.at[pl.ds(sid * _RZ, _RZ)], out_hbm.at[cid, pl.ds(sid * _RZ, _RZ)]
        )

    @pl.when(sid == last_full)
    def _():
        pltpu.sync_copy(
            acc.at[pl.ds(last_full * _RZ, rem)],
            out_hbm.at[cid, pl.ds(last_full * _RZ, rem)],
        )


@functools.cache
def _sc_scatter():
    return pl.kernel(
        _sc_body,
        out_type=jax.ShapeDtypeStruct((_NC, _N, _D), jnp.float32),
        mesh=plsc.VectorSubcoreMesh(
            core_axis_name="c", subcore_axis_name="s", num_cores=_NC, num_subcores=_NS
        ),
        compiler_params=pltpu.CompilerParams(needs_layout_passes=False),
        scratch_types=[
            pltpu.VMEM((_NP, 2, _C), jnp.int32),
            pltpu.VMEM((_NP, _C), jnp.float32),
            pltpu.VMEM((_NR, _C, _D), jnp.float32),
            pltpu.VMEM_SHARED((_NPAD, _D), jnp.float32),
        ] + [pltpu.SemaphoreType.DMA] * (_NP + 2 * _NR),
    )


def _add_body(p_ref, o_ref):
    o_ref[...] = p_ref[0] + p_ref[1]


def _merge_partials(partials):
    ba = 1000
    return pl.pallas_call(
        _add_body,
        grid=(_N // ba,),
        in_specs=[pl.BlockSpec((_NC, ba, _D), lambda i: (0, i, 0))],
        out_specs=pl.BlockSpec((ba, _D), lambda i: (i, 0)),
        out_shape=jax.ShapeDtypeStruct((_N, _D), jnp.float32),
    )(partials)


def kernel(node_features, edge_index, edge_type, edge_norm, weight):
    src = edge_index[0]
    dst = edge_index[1]
    nchunks = _EPAD // _C
    npad = _EPAD - _E
    g = edge_type.astype(jnp.int32) * _N + src.astype(jnp.int32)
    g = jnp.concatenate([g, jnp.zeros((npad,), jnp.int32)])
    d = jnp.concatenate(
        [dst.astype(jnp.int32), _N + jnp.arange(npad, dtype=jnp.int32) % (_NPAD - _N)]
    )
    nrm = jnp.concatenate([edge_norm.astype(jnp.float32), jnp.zeros((npad,), jnp.float32)])
    pack = jnp.stack([g.reshape(nchunks, _C), d.reshape(nchunks, _C)], axis=1)
    normc = nrm.reshape(nchunks, _C)
    y = _relation_matmul(node_features, weight).reshape(_R * _N, _D)
    zeros = jnp.zeros((_RZ, _D), jnp.float32)
    partials = _sc_scatter()(pack, normc, y, zeros)
    return _merge_partials(partials)


# speedup vs baseline: 1.1630x; 1.1630x over previous
import functools

import jax
import jax.numpy as jnp
from jax import lax
from jax.experimental import pallas as pl
from jax.experimental.pallas import tpu as pltpu
from jax.experimental.pallas import tpu_sc as plsc

_N = 10000
_E = 320000
_D = 128
_R = 16

_NC = 2
_NS = 16
_NW = _NC * _NS
_C = 112
_NCHUNK = 90
_K0 = 132
_K1 = 2 * _NCHUNK - _K0
_EPW = _NCHUNK * _C
_EPAD = 2 * _NCHUNK * _C * _NS
_NR = 3
_NP = 6
_NPAD = 10112
_RZ = _NPAD // _NS


def _mm_body(h_ref, w_ref, y_ref):
    y_ref[0] = jnp.dot(h_ref[...], w_ref[0], preferred_element_type=jnp.float32)


def _relation_matmul(node_features, weight):
    bn = 2000
    return pl.pallas_call(
        _mm_body,
        grid=(_N // bn, _R),
        in_specs=[
            pl.BlockSpec((bn, _D), lambda i, r: (i, 0)),
            pl.BlockSpec((1, _D, _D), lambda i, r: (r, 0, 0)),
        ],
        out_specs=pl.BlockSpec((1, bn, _D), lambda i, r: (r, i, 0)),
        out_shape=jax.ShapeDtypeStruct((_R, _N, _D), jnp.float32),
    )(node_features, weight)


def _sc_body(pack_hbm, norm_hbm, y_hbm, zeros_hbm, out_hbm,
             pack_v, norm_v, rows_v, acc, *sems):
    cid = lax.axis_index("c")
    sid = lax.axis_index("s")
    base = sid * (2 * _NCHUNK) + cid * _K0
    nself = lax.select(cid == 0, jnp.int32(_K0), jnp.int32(_K1))
    lsem = sems[0:_NP]
    gsem = sems[_NP:_NP + _NR]
    ssem = sems[_NP + _NR:_NP + 2 * _NR]

    pltpu.sync_copy(zeros_hbm, acc.at[pl.ds(sid * _RZ, _RZ)])
    plsc.subcore_barrier()

    def start_load(p, k):
        pltpu.async_copy(pack_hbm.at[base + k], pack_v.at[p], lsem[p])
        pltpu.async_copy(norm_hbm.at[base + k], norm_v.at[p], lsem[p])

    def wait_load(p):
        pltpu.make_async_copy(pack_hbm.at[0], pack_v.at[p], lsem[p]).wait()
        pltpu.make_async_copy(norm_hbm.at[0], norm_v.at[p], lsem[p]).wait()

    def start_gather(s, p):
        pltpu.async_copy(y_hbm.at[pack_v.at[p, 0]], rows_v.at[s], gsem[s])

    def wait_gather(s):
        pltpu.make_async_copy(y_hbm.at[pl.ds(0, _C)], rows_v.at[s], gsem[s]).wait()

    def start_scatter(s, p):
        pltpu.async_copy(rows_v.at[s], acc.at[pack_v.at[p, 1]], ssem[s], add=True)

    def wait_scatter(s):
        pltpu.make_async_copy(y_hbm.at[pl.ds(0, _C)], rows_v.at[s], ssem[s]).wait()

    def scale(s, p):
        def scale_body(i, carry):
            for u in range(4):
                e = i * 4 + u
                nv = plsc.load_gather(
                    norm_v,
                    [jnp.full((16,), p, jnp.int32), jnp.full((16,), e, jnp.int32)],
                )
                for j in range(_D // 16):
                    sl = pl.ds(j * 16, 16)
                    rows_v[s, e, sl] = rows_v[s, e, sl] * nv
            return carry

        lax.fori_loop(0, _C // 4, scale_body, 0)

    for k in range(4):
        start_load(k, k)
    wait_load(0)
    start_gather(0, 0)

    for k in range(6):
        s, p = k % _NR, k % _NP
        s1, p1 = (k + 1) % _NR, (k + 1) % _NP
        if k >= 2:
            wait_scatter(s1)
        wait_load(p1)
        start_gather(s1, p1)
        start_load((k + 4) % _NP, k + 4)
        wait_gather(s)
        scale(s, p)
        start_scatter(s, p)

    def six_body(q, carry):
        for u in range(6):
            k = q * 6 + u
            s, p = u % _NR, u
            s1, p1 = (u + 1) % _NR, (u + 1) % _NP

            @pl.when(k + 1 < nself)
            def _():
                wait_scatter(s1)
                wait_load(p1)
                start_gather(s1, p1)

            @pl.when(k + 4 < nself)
            def _():
                start_load((u + 4) % _NP, k + 4)

            wait_gather(s)
            scale(s, p)
            start_scatter(s, p)
        return carry

    lax.fori_loop(1, nself // 6, six_body, 0)

    for s in range(_NR):
        wait_scatter(s)

    plsc.subcore_barrier()

    last_full = _N // _RZ
    rem = _N - last_full * _RZ

    @pl.when(sid < last_full)
    def _():
        pltpu.sync_copy(
            acc.at[pl.ds(sid * _RZ, _RZ)], out_hbm.at[cid, pl.ds(sid * _RZ, _RZ)]
        )

    @pl.when(sid == last_full)
    def _():
        pltpu.sync_copy(
            acc.at[pl.ds(last_full * _RZ, rem)],
            out_hbm.at[cid, pl.ds(last_full * _RZ, rem)],
        )


@functools.cache
def _sc_scatter():
    return pl.kernel(
        _sc_body,
        out_type=jax.ShapeDtypeStruct((_NC, _N, _D), jnp.float32),
        mesh=plsc.VectorSubcoreMesh(
            core_axis_name="c", subcore_axis_name="s", num_cores=_NC, num_subcores=_NS
        ),
        compiler_params=pltpu.CompilerParams(needs_layout_passes=False),
        scratch_types=[
            pltpu.VMEM((_NP, 2, _C), jnp.int32),
            pltpu.VMEM((_NP, _C), jnp.float32),
            pltpu.VMEM((_NR, _C, _D), jnp.float32),
            pltpu.VMEM_SHARED((_NPAD, _D), jnp.float32),
        ] + [pltpu.SemaphoreType.DMA] * (_NP + 2 * _NR),
    )


def _add_body(p_ref, o_ref):
    o_ref[...] = p_ref[0] + p_ref[1]


def _merge_partials(partials):
    ba = 1000
    return pl.pallas_call(
        _add_body,
        grid=(_N // ba,),
        in_specs=[pl.BlockSpec((_NC, ba, _D), lambda i: (0, i, 0))],
        out_specs=pl.BlockSpec((ba, _D), lambda i: (i, 0)),
        out_shape=jax.ShapeDtypeStruct((_N, _D), jnp.float32),
    )(partials)


def kernel(node_features, edge_index, edge_type, edge_norm, weight):
    src = edge_index[0]
    dst = edge_index[1]
    nchunks = _EPAD // _C
    npad = _EPAD - _E
    g = edge_type.astype(jnp.int32) * _N + src.astype(jnp.int32)
    g = jnp.concatenate([g, jnp.zeros((npad,), jnp.int32)])
    d = jnp.concatenate(
        [dst.astype(jnp.int32), _N + jnp.arange(npad, dtype=jnp.int32) % (_NPAD - _N)]
    )
    nrm = jnp.concatenate([edge_norm.astype(jnp.float32), jnp.zeros((npad,), jnp.float32)])
    pack = jnp.stack([g.reshape(nchunks, _C), d.reshape(nchunks, _C)], axis=1)
    normc = nrm.reshape(nchunks, _C)
    y = _relation_matmul(node_features, weight).reshape(_R * _N, _D)
    zeros = jnp.zeros((_RZ, _D), jnp.float32)
    partials = _sc_scatter()(pack, normc, y, zeros)
    return _merge_partials(partials)


# speedup vs baseline: 1.1829x; 1.0171x over previous
import functools

import jax
import jax.numpy as jnp
from jax import lax
from jax.experimental import pallas as pl
from jax.experimental.pallas import tpu as pltpu
from jax.experimental.pallas import tpu_sc as plsc

_N = 10000
_E = 320000
_D = 128
_R = 16

_NC = 2
_NS = 16
_NW = _NC * _NS
_C = 112
_NCHUNK = 90
_K0 = 132
_K1 = 2 * _NCHUNK - _K0
_EPW = _NCHUNK * _C
_EPAD = 2 * _NCHUNK * _C * _NS
_NR = 3
_NP = 6
_NPAD = 10112
_RZ = _NPAD // _NS


def _mm_body(h_ref, w_ref, y_ref):
    y_ref[0] = jnp.dot(h_ref[...], w_ref[0], preferred_element_type=jnp.float32)


def _relation_matmul(node_features, weight):
    bn = 2000
    return pl.pallas_call(
        _mm_body,
        grid=(_N // bn, _R),
        in_specs=[
            pl.BlockSpec((bn, _D), lambda i, r: (i, 0)),
            pl.BlockSpec((1, _D, _D), lambda i, r: (r, 0, 0)),
        ],
        out_specs=pl.BlockSpec((1, bn, _D), lambda i, r: (r, i, 0)),
        out_shape=jax.ShapeDtypeStruct((_R, _N, _D), jnp.float32),
    )(node_features, weight)


def _sc_body(g_hbm, d_hbm, norm_hbm, y_hbm, zeros_hbm, out_hbm,
             g_v, d_v, norm_v, rows_v, acc, *sems):
    cid = lax.axis_index("c")
    sid = lax.axis_index("s")
    base = sid * (2 * _NCHUNK) + cid * _K0
    nself = lax.select(cid == 0, jnp.int32(_K0), jnp.int32(_K1))
    lsem = sems[0:_NP]
    gsem = sems[_NP:_NP + _NR]
    ssem = sems[_NP + _NR:_NP + 2 * _NR]

    pltpu.sync_copy(zeros_hbm, acc.at[pl.ds(sid * _RZ, _RZ)])
    plsc.subcore_barrier()

    def start_load(p, k):
        pltpu.async_copy(g_hbm.at[base + k], g_v.at[p], lsem[p])
        pltpu.async_copy(d_hbm.at[base + k], d_v.at[p], lsem[p])
        pltpu.async_copy(norm_hbm.at[base + k], norm_v.at[p], lsem[p])

    def wait_load(p):
        pltpu.make_async_copy(g_hbm.at[0], g_v.at[p], lsem[p]).wait()
        pltpu.make_async_copy(d_hbm.at[0], d_v.at[p], lsem[p]).wait()
        pltpu.make_async_copy(norm_hbm.at[0], norm_v.at[p], lsem[p]).wait()

    def start_gather(s, p):
        pltpu.async_copy(y_hbm.at[g_v.at[p]], rows_v.at[s], gsem[s])

    def wait_gather(s):
        pltpu.make_async_copy(y_hbm.at[pl.ds(0, _C)], rows_v.at[s], gsem[s]).wait()

    def start_scatter(s, p):
        pltpu.async_copy(rows_v.at[s], acc.at[d_v.at[p]], ssem[s], add=True)

    def wait_scatter(s):
        pltpu.make_async_copy(y_hbm.at[pl.ds(0, _C)], rows_v.at[s], ssem[s]).wait()

    def scale(s, p):
        def scale_body(i, carry):
            for u in range(4):
                e = i * 4 + u
                nv = plsc.load_gather(
                    norm_v,
                    [jnp.full((16,), p, jnp.int32), jnp.full((16,), e, jnp.int32)],
                )
                for j in range(_D // 16):
                    sl = pl.ds(j * 16, 16)
                    rows_v[s, e, sl] = rows_v[s, e, sl] * nv
            return carry

        lax.fori_loop(0, _C // 4, scale_body, 0)

    for k in range(4):
        start_load(k, k)
    wait_load(0)
    start_gather(0, 0)

    for k in range(6):
        s, p = k % _NR, k % _NP
        s1, p1 = (k + 1) % _NR, (k + 1) % _NP
        if k >= 2:
            wait_scatter(s1)
        wait_load(p1)
        start_gather(s1, p1)
        start_load((k + 4) % _NP, k + 4)
        wait_gather(s)
        scale(s, p)
        start_scatter(s, p)

    def six_body(q, carry):
        for u in range(6):
            k = q * 6 + u
            s, p = u % _NR, u
            s1, p1 = (u + 1) % _NR, (u + 1) % _NP

            @pl.when(k + 1 < nself)
            def _():
                wait_scatter(s1)
                wait_load(p1)
                start_gather(s1, p1)

            @pl.when(k + 4 < nself)
            def _():
                start_load((u + 4) % _NP, k + 4)

            wait_gather(s)
            scale(s, p)
            start_scatter(s, p)
        return carry

    lax.fori_loop(1, nself // 6, six_body, 0)

    for s in range(_NR):
        wait_scatter(s)

    plsc.subcore_barrier()

    last_full = _N // _RZ
    rem = _N - last_full * _RZ

    @pl.when(sid < last_full)
    def _():
        pltpu.sync_copy(
            acc.at[pl.ds(sid * _RZ, _RZ)], out_hbm.at[cid, pl.ds(sid * _RZ, _RZ)]
        )

    @pl.when(sid == last_full)
    def _():
        pltpu.sync_copy(
            acc.at[pl.ds(last_full * _RZ, rem)],
            out_hbm.at[cid, pl.ds(last_full * _RZ, rem)],
        )


@functools.cache
def _sc_scatter():
    return pl.kernel(
        _sc_body,
        out_type=jax.ShapeDtypeStruct((_NC, _N, _D), jnp.float32),
        mesh=plsc.VectorSubcoreMesh(
            core_axis_name="c", subcore_axis_name="s", num_cores=_NC, num_subcores=_NS
        ),
        compiler_params=pltpu.CompilerParams(needs_layout_passes=False),
        scratch_types=[
            pltpu.VMEM((_NP, _C), jnp.int32),
            pltpu.VMEM((_NP, _C), jnp.int32),
            pltpu.VMEM((_NP, _C), jnp.float32),
            pltpu.VMEM((_NR, _C, _D), jnp.float32),
            pltpu.VMEM_SHARED((_NPAD, _D), jnp.float32),
        ] + [pltpu.SemaphoreType.DMA] * (_NP + 2 * _NR),
    )


def _add_body(p_ref, o_ref):
    o_ref[...] = p_ref[0] + p_ref[1]


def _merge_partials(partials):
    ba = 1000
    return pl.pallas_call(
        _add_body,
        grid=(_N // ba,),
        in_specs=[pl.BlockSpec((_NC, ba, _D), lambda i: (0, i, 0))],
        out_specs=pl.BlockSpec((ba, _D), lambda i: (i, 0)),
        out_shape=jax.ShapeDtypeStruct((_N, _D), jnp.float32),
    )(partials)


def kernel(node_features, edge_index, edge_type, edge_norm, weight):
    src = edge_index[0]
    dst = edge_index[1]
    nchunks = _EPAD // _C
    npad = _EPAD - _E
    g = edge_type.astype(jnp.int32) * _N + src.astype(jnp.int32)
    g = jnp.concatenate([g, jnp.zeros((npad,), jnp.int32)])
    d = jnp.concatenate(
        [dst.astype(jnp.int32), _N + jnp.arange(npad, dtype=jnp.int32) % (_NPAD - _N)]
    )
    nrm = jnp.concatenate([edge_norm.astype(jnp.float32), jnp.zeros((npad,), jnp.float32)])
    normc = nrm.reshape(nchunks, _C)
    y = _relation_matmul(node_features, weight).reshape(_R * _N, _D)
    zeros = jnp.zeros((_RZ, _D), jnp.float32)
    partials = _sc_scatter()(g.reshape(nchunks, _C), d.reshape(nchunks, _C), normc, y, zeros)
    return _merge_partials(partials)
